# Initial kernel scaffold; baseline (speedup 1.0000x reference)
import functools

import jax
import jax.numpy as jnp
from jax import lax
from jax.experimental import pallas as pl
from jax.experimental.pallas import tpu as pltpu
from jax.experimental.pallas import tpu_sc as plsc

NUM_STAGES = 4
K = 8192
D = 256
N_TOK = 4608

M_TILE = 512
N_TILE = 2048
N_M = N_TOK // M_TILE
N_N = K // N_TILE

_NC = 2
_NS = 16
_NW = _NC * _NS
_BPW = N_TOK // _NW
_CH = 72


def _argmin_body(cur_ref, cb_ref, cn_ref, enc_ref, minv, mini):
    c = pl.program_id(1)
    cur = cur_ref[...]
    cb = cb_ref[...]
    rn = jnp.sum(cur * cur, axis=1, keepdims=True)
    mm = lax.dot_general(cur, cb, (((1,), (1,)), ((), ())),
                         preferred_element_type=jnp.float32)
    d2 = rn - 2.0 * mm + cn_ref[...]
    dist = jnp.sqrt(jnp.maximum(d2, 0.0))
    bmin = jnp.min(dist, axis=1, keepdims=True)
    iota = lax.broadcasted_iota(jnp.int32, dist.shape, 1)
    bidx = jnp.min(jnp.where(dist == bmin, iota, jnp.int32(K)),
                   axis=1, keepdims=True) + c * N_TILE

    @pl.when(c == 0)
    def _():
        minv[...] = bmin
        mini[...] = bidx

    @pl.when(c > 0)
    def _():
        upd = bmin < minv[...]
        minv[...] = jnp.where(upd, bmin, minv[...])
        mini[...] = jnp.where(upd, bidx, mini[...])

    @pl.when(c == N_N - 1)
    def _():
        enc_ref[...] = jnp.broadcast_to(mini[...], (M_TILE, 128))


def _stage_argmin(cur, cb, cn_row):
    enc2d = pl.pallas_call(
        _argmin_body,
        grid=(N_M, N_N),
        in_specs=[
            pl.BlockSpec((M_TILE, D), lambda t, c: (t, 0)),
            pl.BlockSpec((N_TILE, D), lambda t, c: (c, 0)),
            pl.BlockSpec((1, N_TILE), lambda t, c: (0, c)),
        ],
        out_specs=pl.BlockSpec((M_TILE, 128), lambda t, c: (t, 0)),
        out_shape=jax.ShapeDtypeStruct((N_TOK, 128), jnp.int32),
        scratch_shapes=[
            pltpu.VMEM((M_TILE, 1), jnp.float32),
            pltpu.VMEM((M_TILE, 1), jnp.int32),
        ],
    )(cur, cb, cn_row)
    return enc2d[:, 0]


@functools.partial(
    pl.kernel,
    mesh=plsc.VectorSubcoreMesh(core_axis_name="c", subcore_axis_name="s"),
    out_type=jax.ShapeDtypeStruct((N_TOK, D), jnp.float32),
    scratch_types=[
        pltpu.VMEM((_CH,), jnp.int32),
        pltpu.VMEM((_CH,), jnp.int32),
        pltpu.VMEM((_BPW, D), jnp.float32),
        pltpu.SemaphoreType.DMA,
    ],
)
def _sc_gather(table_hbm, idx_hbm, out_hbm, idx_a, idx_b, rows_v, sem):
    wid = lax.axis_index("s") * _NC + lax.axis_index("c")
    base = wid * _BPW
    pltpu.sync_copy(idx_hbm.at[pl.ds(base, _CH)], idx_a)
    pltpu.sync_copy(idx_hbm.at[pl.ds(base + _CH, _CH)], idx_b)
    pltpu.async_copy(table_hbm.at[idx_a], rows_v.at[pl.ds(0, _CH)], sem).wait()
    pltpu.async_copy(table_hbm.at[idx_b], rows_v.at[pl.ds(_CH, _CH)], sem).wait()
    pltpu.sync_copy(rows_v, out_hbm.at[pl.ds(base, _BPW)])


def kernel(x, codebooks):
    b, s, d = x.shape
    cur = x.reshape(-1, d)
    cn = jnp.sum(codebooks * codebooks, axis=2)
    quant = jnp.zeros_like(cur)
    loss_cd = jnp.zeros((), dtype=jnp.float32)
    loss_enc = jnp.zeros((), dtype=jnp.float32)
    encs = []
    for i in range(NUM_STAGES):
        enc = _stage_argmin(cur, codebooks[i], cn[i][None, :])
        nearest = _sc_gather(codebooks[i], enc)
        loss_i = jnp.mean((cur - nearest) ** 2)
        loss_cd = loss_cd + loss_i
        loss_enc = loss_enc + loss_i
        cur = cur - nearest
        quant = quant + nearest
        encs.append(enc)
    discrete_enc = jnp.stack(encs, axis=-1).reshape(b, s, NUM_STAGES)
    quantised = (cur + (quant - cur)).reshape(b, s, d)
    return (loss_cd, loss_enc, discrete_enc, quantised)


# speedup vs baseline: 1.0282x; 1.0282x over previous
import functools

import jax
import jax.numpy as jnp
from jax import lax
from jax.experimental import pallas as pl
from jax.experimental.pallas import tpu as pltpu
from jax.experimental.pallas import tpu_sc as plsc

NUM_STAGES = 4
K = 8192
D = 256
N_TOK = 4608

M_TILE = 512
N_TILE = 2048
N_M = N_TOK // M_TILE
N_N = K // N_TILE

_NC = 2
_NS = 16
_NW = _NC * _NS
_BPW = N_TOK // _NW
_CH = 72


def _argmin_body(cur_ref, cb_ref, cn_ref, enc_ref, minv, mini):
    c = pl.program_id(1)
    cur = cur_ref[...]
    cb = cb_ref[...]
    rn = jnp.sum(cur * cur, axis=1, keepdims=True)
    mm = lax.dot_general(cur, cb, (((1,), (1,)), ((), ())),
                         preferred_element_type=jnp.float32)
    d2 = rn - 2.0 * mm + cn_ref[...]
    dist = jnp.sqrt(jnp.maximum(d2, 0.0))
    bmin = jnp.min(dist, axis=1, keepdims=True)
    iota = lax.broadcasted_iota(jnp.int32, dist.shape, 1)
    bidx = jnp.min(jnp.where(dist == bmin, iota, jnp.int32(K)),
                   axis=1, keepdims=True) + c * N_TILE

    @pl.when(c == 0)
    def _():
        minv[...] = bmin
        mini[...] = bidx

    @pl.when(c > 0)
    def _():
        upd = bmin < minv[...]
        minv[...] = jnp.where(upd, bmin, minv[...])
        mini[...] = jnp.where(upd, bidx, mini[...])

    @pl.when(c == N_N - 1)
    def _():
        enc_ref[...] = jnp.broadcast_to(mini[...], (M_TILE, 128))


def _stage_argmin(cur, cb, cn_row):
    enc2d = pl.pallas_call(
        _argmin_body,
        grid=(N_M, N_N),
        in_specs=[
            pl.BlockSpec((M_TILE, D), lambda t, c: (t, 0)),
            pl.BlockSpec((N_TILE, D), lambda t, c: (c, 0)),
            pl.BlockSpec((1, N_TILE), lambda t, c: (0, c)),
        ],
        out_specs=pl.BlockSpec((M_TILE, 128), lambda t, c: (t, 0)),
        out_shape=jax.ShapeDtypeStruct((N_TOK, 128), jnp.int32),
        scratch_shapes=[
            pltpu.VMEM((M_TILE, 1), jnp.float32),
            pltpu.VMEM((M_TILE, 1), jnp.int32),
        ],
    )(cur, cb, cn_row)
    return enc2d[:, 0]


def _sc_gather_body(table_hbm, idx_hbm, out_hbm, idx_a, idx_b, rows_v, sem):
    wid = lax.axis_index("s") * _NC + lax.axis_index("c")
    base = wid * _BPW
    pltpu.sync_copy(idx_hbm.at[pl.ds(base, _CH)], idx_a)
    pltpu.sync_copy(idx_hbm.at[pl.ds(base + _CH, _CH)], idx_b)
    pltpu.async_copy(table_hbm.at[idx_a], rows_v.at[pl.ds(0, _CH)], sem).wait()
    pltpu.async_copy(table_hbm.at[idx_b], rows_v.at[pl.ds(_CH, _CH)], sem).wait()
    pltpu.sync_copy(rows_v, out_hbm.at[pl.ds(base, _BPW)])


@functools.cache
def _sc_gather():
    return pl.kernel(
        _sc_gather_body,
        mesh=plsc.VectorSubcoreMesh(core_axis_name="c", subcore_axis_name="s"),
        out_type=jax.ShapeDtypeStruct((N_TOK, D), jnp.float32),
        scratch_types=[
            pltpu.VMEM((_CH,), jnp.int32),
            pltpu.VMEM((_CH,), jnp.int32),
            pltpu.VMEM((_BPW, D), jnp.float32),
            pltpu.SemaphoreType.DMA,
        ],
    )


def kernel(x, codebooks):
    b, s, d = x.shape
    cur = x.reshape(-1, d)
    cn = jnp.sum(codebooks * codebooks, axis=2)
    quant = jnp.zeros_like(cur)
    loss_cd = jnp.zeros((), dtype=jnp.float32)
    loss_enc = jnp.zeros((), dtype=jnp.float32)
    encs = []
    for i in range(NUM_STAGES):
        enc = _stage_argmin(cur, codebooks[i], cn[i][None, :])
        nearest = _sc_gather()(codebooks[i], enc)
        loss_i = jnp.mean((cur - nearest) ** 2)
        loss_cd = loss_cd + loss_i
        loss_enc = loss_enc + loss_i
        cur = cur - nearest
        quant = quant + nearest
        encs.append(enc)
    discrete_enc = jnp.stack(encs, axis=-1).reshape(b, s, NUM_STAGES)
    quantised = (cur + (quant - cur)).reshape(b, s, d)
    return (loss_cd, loss_enc, discrete_enc, quantised)


# speedup vs baseline: 1.4944x; 1.4534x over previous
import functools

import jax
import jax.numpy as jnp
from jax import lax
from jax.experimental import pallas as pl
from jax.experimental.pallas import tpu as pltpu
from jax.experimental.pallas import tpu_sc as plsc

NUM_STAGES = 4
K = 8192
D = 256
N_TOK = 4608

M_TILE = 512
N_TILE = 2048
N_M = N_TOK // M_TILE
N_N = K // N_TILE

_NC = 2
_NS = 16
_NW = _NC * _NS
_BPW = N_TOK // _NW
_CH = 72


def _argmin_body(cur2_ref, cb_ref, cn_ref, enc_ref, d2_ref):
    cur2 = cur2_ref[...]
    rn = 0.25 * jnp.sum(cur2 * cur2, axis=1, keepdims=True)
    dmin = None
    for c in range(N_N):
        cb = cb_ref[pl.ds(c * N_TILE, N_TILE), :]
        mm2 = lax.dot_general(cur2, cb, (((1,), (1,)), ((), ())),
                              preferred_element_type=jnp.float32)
        d2 = (rn + mm2) + cn_ref[:, pl.ds(c * N_TILE, N_TILE)]
        d2_ref[:, pl.ds(c * N_TILE, N_TILE)] = d2
        rmin = jnp.min(d2, axis=1, keepdims=True)
        dmin = rmin if c == 0 else jnp.minimum(dmin, rmin)
    mstar = jnp.sqrt(jnp.maximum(dmin, 0.0))
    bits = lax.bitcast_convert_type(dmin, jnp.int32)
    pos = dmin > 0.0
    bb = jnp.where(pos, dmin, 0.0)
    for k in range(1, 7):
        cand = lax.bitcast_convert_type(bits + k, jnp.float32)
        ok = pos & (jnp.sqrt(cand) == mstar)
        bb = jnp.where(ok, cand, bb)
    kmin = None
    for c in range(N_N):
        d2 = d2_ref[:, pl.ds(c * N_TILE, N_TILE)]
        io = lax.broadcasted_iota(jnp.int32, (M_TILE, N_TILE), 1) + c * N_TILE
        key = jnp.where(d2 <= bb, io, jnp.int32(K))
        r = jnp.min(key, axis=1, keepdims=True)
        kmin = r if c == 0 else jnp.minimum(kmin, r)
    enc_ref[...] = jnp.broadcast_to(kmin, (M_TILE, 128))


def _stage_argmin(cur2, cb, cn_row):
    enc2d = pl.pallas_call(
        _argmin_body,
        grid=(N_M,),
        in_specs=[
            pl.BlockSpec((M_TILE, D), lambda t: (t, 0)),
            pl.BlockSpec((K, D), lambda t: (0, 0)),
            pl.BlockSpec((1, K), lambda t: (0, 0)),
        ],
        out_specs=pl.BlockSpec((M_TILE, 128), lambda t: (t, 0)),
        out_shape=jax.ShapeDtypeStruct((N_TOK, 128), jnp.int32),
        scratch_shapes=[
            pltpu.VMEM((M_TILE, K), jnp.float32),
        ],
    )(cur2, cb, cn_row)
    return enc2d[:, 0]


def _sc_gather_body(table_hbm, idx_hbm, out_hbm, idx_a, idx_b, rows_v, sem):
    wid = lax.axis_index("s") * _NC + lax.axis_index("c")
    base = wid * _BPW
    pltpu.sync_copy(idx_hbm.at[pl.ds(base, _CH)], idx_a)
    pltpu.sync_copy(idx_hbm.at[pl.ds(base + _CH, _CH)], idx_b)
    pltpu.async_copy(table_hbm.at[idx_a], rows_v.at[pl.ds(0, _CH)], sem).wait()
    pltpu.async_copy(table_hbm.at[idx_b], rows_v.at[pl.ds(_CH, _CH)], sem).wait()
    pltpu.sync_copy(rows_v, out_hbm.at[pl.ds(base, _BPW)])


@functools.cache
def _sc_gather():
    return pl.kernel(
        _sc_gather_body,
        mesh=plsc.VectorSubcoreMesh(core_axis_name="c", subcore_axis_name="s"),
        out_type=jax.ShapeDtypeStruct((N_TOK, D), jnp.float32),
        scratch_types=[
            pltpu.VMEM((_CH,), jnp.int32),
            pltpu.VMEM((_CH,), jnp.int32),
            pltpu.VMEM((_BPW, D), jnp.float32),
            pltpu.SemaphoreType.DMA,
        ],
    )


def kernel(x, codebooks):
    b, s, d = x.shape
    cur = x.reshape(-1, d)
    cn = jnp.sum(codebooks * codebooks, axis=2)
    quant = jnp.zeros_like(cur)
    loss_cd = jnp.zeros((), dtype=jnp.float32)
    loss_enc = jnp.zeros((), dtype=jnp.float32)
    encs = []
    for i in range(NUM_STAGES):
        enc = _stage_argmin(-2.0 * cur, codebooks[i], cn[i][None, :])
        nearest = _sc_gather()(codebooks[i], enc)
        loss_i = jnp.mean((cur - nearest) ** 2)
        loss_cd = loss_cd + loss_i
        loss_enc = loss_enc + loss_i
        cur = cur - nearest
        quant = quant + nearest
        encs.append(enc)
    discrete_enc = jnp.stack(encs, axis=-1).reshape(b, s, NUM_STAGES)
    quantised = (cur + (quant - cur)).reshape(b, s, d)
    return (loss_cd, loss_enc, discrete_enc, quantised)


# speedup vs baseline: 1.8025x; 1.2061x over previous
import functools

import jax
import jax.numpy as jnp
from jax import lax
from jax.experimental import pallas as pl
from jax.experimental.pallas import tpu as pltpu
from jax.experimental.pallas import tpu_sc as plsc

NUM_STAGES = 4
K = 8192
D = 256
N_TOK = 4608

M_TILE = 512
N_TILE = 2048
N_M = N_TOK // M_TILE
N_N = K // N_TILE

_NC = 2
_NS = 16
_NW = _NC * _NS
_BPW = N_TOK // _NW
_CH = 72

_INV_N = 1.0 / (N_TOK * D)


def _fused_argmin(cur2, cb_ref, cn_ref, enc_ref, d2_ref):
    rn = 0.25 * jnp.sum(cur2 * cur2, axis=1, keepdims=True)
    dmin = None
    for c in range(N_N):
        cb = cb_ref[pl.ds(c * N_TILE, N_TILE), :]
        mm2 = lax.dot_general(cur2, cb, (((1,), (1,)), ((), ())),
                              preferred_element_type=jnp.float32)
        d2 = (rn + mm2) + cn_ref[:, pl.ds(c * N_TILE, N_TILE)]
        d2_ref[:, pl.ds(c * N_TILE, N_TILE)] = d2
        rmin = jnp.min(d2, axis=1, keepdims=True)
        dmin = rmin if c == 0 else jnp.minimum(dmin, rmin)
    mstar = jnp.sqrt(jnp.maximum(dmin, 0.0))
    bits = lax.bitcast_convert_type(dmin, jnp.int32)
    pos = dmin > 0.0
    bb = jnp.where(pos, dmin, 0.0)
    for k in range(1, 7):
        cand = lax.bitcast_convert_type(bits + k, jnp.float32)
        ok = pos & (jnp.sqrt(cand) == mstar)
        bb = jnp.where(ok, cand, bb)
    BIAS = 0x4B000000
    kmin = None
    for c in range(N_N):
        d2 = d2_ref[:, pl.ds(c * N_TILE, N_TILE)]
        io = lax.broadcasted_iota(jnp.int32, (M_TILE, N_TILE), 1) + (BIAS + c * N_TILE)
        key = lax.bitcast_convert_type(jnp.where(d2 <= bb, io, jnp.int32(BIAS + K)),
                                       jnp.float32)
        r = jnp.min(key, axis=1, keepdims=True)
        kmin = r if c == 0 else jnp.minimum(kmin, r)
    idx = lax.bitcast_convert_type(kmin, jnp.int32) - BIAS
    enc_ref[...] = jnp.broadcast_to(idx, (M_TILE, 128))


def _body_first(x_ref, cb_ref, cn_ref, enc_ref, d2_ref):
    _fused_argmin(-2.0 * x_ref[...], cb_ref, cn_ref, enc_ref, d2_ref)


def _body_mid(curp_ref, near_ref, quantp_ref, cb_ref, cn_ref,
              enc_ref, cur_ref, quant_ref, ls_ref, d2_ref):
    cur = curp_ref[...] - near_ref[...]
    cur_ref[...] = cur
    quant_ref[...] = quantp_ref[...] + near_ref[...]
    cur2 = -2.0 * cur
    rn = 0.25 * jnp.sum(cur2 * cur2, axis=1, keepdims=True)
    ls_ref[...] = jnp.broadcast_to(jnp.sum(rn).reshape(1, 1, 1), (1, 1, 128))
    _fused_argmin(cur2, cb_ref, cn_ref, enc_ref, d2_ref)


def _body_last(curp_ref, near_ref, quantp_ref, qout_ref, ls_ref):
    cur = curp_ref[...] - near_ref[...]
    quant = quantp_ref[...] + near_ref[...]
    qout_ref[...] = cur + (quant - cur)
    rn = jnp.sum(cur * cur, axis=1, keepdims=True)
    ls_ref[...] = jnp.broadcast_to(jnp.sum(rn).reshape(1, 1, 1), (1, 1, 128))


_TOK_SPEC = pl.BlockSpec((M_TILE, D), lambda t: (t, 0))
_LS_SPEC = pl.BlockSpec((1, 1, 128), lambda t: (t, 0, 0))


def _stage_first(x2d, cb, cn_row):
    enc2d = pl.pallas_call(
        _body_first,
        grid=(N_M,),
        in_specs=[
            _TOK_SPEC,
            pl.BlockSpec((K, D), lambda t: (0, 0)),
            pl.BlockSpec((1, K), lambda t: (0, 0)),
        ],
        out_specs=pl.BlockSpec((M_TILE, 128), lambda t: (t, 0)),
        out_shape=jax.ShapeDtypeStruct((N_TOK, 128), jnp.int32),
        scratch_shapes=[pltpu.VMEM((M_TILE, K), jnp.float32)],
    )(x2d, cb, cn_row)
    return enc2d[:, 0]


def _stage_mid(curp, near, quantp, cb, cn_row):
    enc2d, cur, quant, ls = pl.pallas_call(
        _body_mid,
        grid=(N_M,),
        in_specs=[
            _TOK_SPEC,
            _TOK_SPEC,
            _TOK_SPEC,
            pl.BlockSpec((K, D), lambda t: (0, 0)),
            pl.BlockSpec((1, K), lambda t: (0, 0)),
        ],
        out_specs=[
            pl.BlockSpec((M_TILE, 128), lambda t: (t, 0)),
            _TOK_SPEC,
            _TOK_SPEC,
            _LS_SPEC,
        ],
        out_shape=[
            jax.ShapeDtypeStruct((N_TOK, 128), jnp.int32),
            jax.ShapeDtypeStruct((N_TOK, D), jnp.float32),
            jax.ShapeDtypeStruct((N_TOK, D), jnp.float32),
            jax.ShapeDtypeStruct((N_M, 1, 128), jnp.float32),
        ],
        scratch_shapes=[pltpu.VMEM((M_TILE, K), jnp.float32)],
    )(curp, near, quantp, cb, cn_row)
    return enc2d[:, 0], cur, quant, ls


def _stage_last(curp, near, quantp):
    qout, ls = pl.pallas_call(
        _body_last,
        grid=(N_M,),
        in_specs=[_TOK_SPEC, _TOK_SPEC, _TOK_SPEC],
        out_specs=[_TOK_SPEC, _LS_SPEC],
        out_shape=[
            jax.ShapeDtypeStruct((N_TOK, D), jnp.float32),
            jax.ShapeDtypeStruct((N_M, 1, 128), jnp.float32),
        ],
    )(curp, near, quantp)
    return qout, ls


def _sc_gather_body(table_hbm, idx_hbm, out_hbm, idx_a, idx_b, rows_v, sem):
    wid = lax.axis_index("s") * _NC + lax.axis_index("c")
    base = wid * _BPW
    pltpu.sync_copy(idx_hbm.at[pl.ds(base, _CH)], idx_a)
    pltpu.sync_copy(idx_hbm.at[pl.ds(base + _CH, _CH)], idx_b)
    pltpu.async_copy(table_hbm.at[idx_a], rows_v.at[pl.ds(0, _CH)], sem).wait()
    pltpu.async_copy(table_hbm.at[idx_b], rows_v.at[pl.ds(_CH, _CH)], sem).wait()
    pltpu.sync_copy(rows_v, out_hbm.at[pl.ds(base, _BPW)])


@functools.cache
def _sc_gather():
    return pl.kernel(
        _sc_gather_body,
        mesh=plsc.VectorSubcoreMesh(core_axis_name="c", subcore_axis_name="s"),
        out_type=jax.ShapeDtypeStruct((N_TOK, D), jnp.float32),
        scratch_types=[
            pltpu.VMEM((_CH,), jnp.int32),
            pltpu.VMEM((_CH,), jnp.int32),
            pltpu.VMEM((_BPW, D), jnp.float32),
            pltpu.SemaphoreType.DMA,
        ],
    )


def kernel(x, codebooks):
    b, s, d = x.shape
    x2d = x.reshape(-1, d)
    cn = jnp.sum(codebooks * codebooks, axis=2)

    enc0 = _stage_first(x2d, codebooks[0], cn[0][None, :])
    near = _sc_gather()(codebooks[0], enc0)
    encs = [enc0]
    cur, quant = x2d, jnp.zeros_like(x2d)
    ls_parts = []
    for i in range(1, NUM_STAGES):
        enc, cur, quant, ls = _stage_mid(cur, near, quant,
                                         codebooks[i], cn[i][None, :])
        near = _sc_gather()(codebooks[i], enc)
        encs.append(enc)
        ls_parts.append(ls)
    quantised, ls = _stage_last(cur, near, quant)
    ls_parts.append(ls)

    loss = jnp.zeros((), dtype=jnp.float32)
    for ls in ls_parts:
        loss = loss + jnp.sum(ls[:, 0, 0]) * _INV_N
    discrete_enc = jnp.stack(encs, axis=-1).reshape(b, s, NUM_STAGES)
    return (loss, loss, discrete_enc, quantised.reshape(b, s, d))
